# Initial kernel scaffold; baseline (speedup 1.0000x reference)
import jax
import jax.numpy as jnp
from jax.experimental import pallas as pl


def kernel(X, node_idx, hedge_idx, W, b):
    raise NotImplementedError("write your pallas kernel here")


# speedup vs baseline: 7.6872x; 7.6872x over previous
import functools

import jax
import jax.numpy as jnp
from jax import lax
from jax.experimental import pallas as pl
from jax.experimental.pallas import tpu as pltpu
from jax.experimental.pallas import tpu_sc as plsc

N_NODES = 10000
N_HEDGES = 5000
NNZ = 320000
D = 128

NC = 2
NS = 16
NW = NC * NS
P = NNZ // NW
C = 80
NCH = P // C

EF_PAD = 5120
OUT_PAD = 10240

@functools.cache
def _mesh():
    return plsc.VectorSubcoreMesh(core_axis_name="c", subcore_axis_name="s",
                                  num_cores=NC, num_subcores=NS)


def _hist_body(ni_hbm, hi_hbm, zeros_hbm, ones_hbm, dv_out, de_out,
               dv_sh, de_sh, idx_v, ones_v):
    c = lax.axis_index("c")
    s = lax.axis_index("s")
    wid = c * NS + s

    @pl.when(s == 0)
    def _():
        pltpu.sync_copy(zeros_hbm, dv_sh)

    @pl.when(s == 1)
    def _():
        pltpu.sync_copy(zeros_hbm.at[pl.ds(0, EF_PAD)], de_sh)

    pltpu.sync_copy(ones_hbm, ones_v)
    pltpu.sync_copy(ni_hbm.at[wid], idx_v)
    plsc.subcore_barrier()

    def nbody(j, carry):
        pltpu.sync_copy(ones_v, dv_sh.at[idx_v.at[j]], add=True)
        return carry

    lax.fori_loop(0, NCH, nbody, 0)

    pltpu.sync_copy(hi_hbm.at[wid], idx_v)

    def hbody(j, carry):
        pltpu.sync_copy(ones_v, de_sh.at[idx_v.at[j]], add=True)
        return carry

    lax.fori_loop(0, NCH, hbody, 0)
    plsc.subcore_barrier()

    @pl.when(s == 0)
    def _():
        pltpu.sync_copy(dv_sh, dv_out.at[c])

    @pl.when(s == 1)
    def _():
        pltpu.sync_copy(de_sh, de_out.at[c])


@functools.cache
def _hist():
    return pl.kernel(
        _hist_body,
        out_type=(jax.ShapeDtypeStruct((NC, N_NODES), jnp.float32),
                  jax.ShapeDtypeStruct((NC, EF_PAD), jnp.float32)),
        mesh=_mesh(),
        scratch_types=[
            pltpu.VMEM_SHARED((N_NODES,), jnp.float32),
            pltpu.VMEM_SHARED((EF_PAD,), jnp.float32),
            pltpu.VMEM((NCH, C), jnp.int32),
            pltpu.VMEM((C,), jnp.float32),
        ],
    )


def _proj_body(x_ref, w_ref, b_ref, dvp_ref, xs_ref, dvis_ref):
    x = x_ref[...]
    w = w_ref[...]
    b = b_ref[...]
    dvp = dvp_ref[...]
    dv = dvp[0] + dvp[1]
    dvis = jnp.where(dv > 0, lax.rsqrt(jnp.maximum(dv, 1e-12)), 0.0)
    xt = jnp.dot(x, w, preferred_element_type=jnp.float32) + b
    xs_ref[...] = xt * dvis[:, None]
    dvis_ref[...] = dvis[None, :]


def _project(X, W, b, dvp):
    return pl.pallas_call(
        _proj_body,
        out_shape=[
            jax.ShapeDtypeStruct((N_NODES, D), jnp.float32),
            jax.ShapeDtypeStruct((1, N_NODES), jnp.float32),
        ],
    )(X, W, b, dvp)


def _edge_body(xs_hbm, ni_hbm, hi_hbm, zeros_hbm, ef_out,
               ef_sh, idxn, idxh, rows, sem):
    c = lax.axis_index("c")
    s = lax.axis_index("s")
    wid = c * NS + s

    pltpu.sync_copy(zeros_hbm, ef_sh.at[pl.ds(s * 320, 320)])
    pltpu.sync_copy(ni_hbm.at[wid], idxn)
    pltpu.sync_copy(hi_hbm.at[wid], idxh)
    plsc.subcore_barrier()

    def body(j, carry):
        pltpu.async_copy(xs_hbm.at[idxn.at[j]], rows, sem).wait()
        pltpu.sync_copy(rows, ef_sh.at[idxh.at[j]], add=True)
        return carry

    lax.fori_loop(0, NCH, body, 0)
    plsc.subcore_barrier()
    pltpu.sync_copy(ef_sh.at[pl.ds(s * 320, 320)],
                    ef_out.at[c, pl.ds(s * 320, 320)])


@functools.cache
def _edge_agg():
    return pl.kernel(
        _edge_body,
        out_type=jax.ShapeDtypeStruct((NC, EF_PAD, D), jnp.float32),
        mesh=_mesh(),
        scratch_types=[
            pltpu.VMEM_SHARED((EF_PAD, D), jnp.float32),
            pltpu.VMEM((NCH, C), jnp.int32),
            pltpu.VMEM((NCH, C), jnp.int32),
            pltpu.VMEM((C, D), jnp.float32),
            pltpu.SemaphoreType.DMA,
        ],
    )


def _escale_body(efp_ref, dep_ref, ef_ref):
    efp = efp_ref[...]
    dep = dep_ref[...]
    de = dep[0] + dep[1]
    dei = jnp.where(de > 0, 1.0 / jnp.maximum(de, 1e-12), 0.0)
    ef_ref[...] = (efp[0] + efp[1]) * dei[:, None]


def _edge_scale(efp, dep):
    return pl.pallas_call(
        _escale_body,
        out_shape=jax.ShapeDtypeStruct((EF_PAD, D), jnp.float32),
    )(efp, dep)


def _node_body(ef_hbm, ni_hbm, hi_hbm, zeros_hbm, out_hbm,
               out_sh, idxn, idxh, rows, sem):
    c = lax.axis_index("c")
    s = lax.axis_index("s")
    wid = c * NS + s

    pltpu.sync_copy(zeros_hbm, out_sh.at[pl.ds(s * 640, 320)])
    pltpu.sync_copy(zeros_hbm, out_sh.at[pl.ds(s * 640 + 320, 320)])
    pltpu.sync_copy(ni_hbm.at[wid], idxn)
    pltpu.sync_copy(hi_hbm.at[wid], idxh)
    plsc.subcore_barrier()

    def body(j, carry):
        pltpu.async_copy(ef_hbm.at[idxh.at[j]], rows, sem).wait()
        pltpu.sync_copy(rows, out_sh.at[idxn.at[j]], add=True)
        return carry

    lax.fori_loop(0, NCH, body, 0)
    plsc.subcore_barrier()
    pltpu.sync_copy(out_sh.at[pl.ds(s * 640, 640)],
                    out_hbm.at[c, pl.ds(s * 640, 640)])


@functools.cache
def _node_agg():
    return pl.kernel(
        _node_body,
        out_type=jax.ShapeDtypeStruct((NC, OUT_PAD, D), jnp.float32),
        mesh=_mesh(),
        scratch_types=[
            pltpu.VMEM_SHARED((OUT_PAD, D), jnp.float32),
            pltpu.VMEM((NCH, C), jnp.int32),
            pltpu.VMEM((NCH, C), jnp.int32),
            pltpu.VMEM((C, D), jnp.float32),
            pltpu.SemaphoreType.DMA,
        ],
    )


def _final_body(op_ref, dvis_ref, o_ref):
    op = op_ref[...]
    dvis = dvis_ref[...]
    o_ref[...] = jnp.maximum((op[0] + op[1]) * dvis[0][:, None], 0.0)


def _finalize(outp, dvis):
    return pl.pallas_call(
        _final_body,
        grid=(1,),
        in_specs=[
            pl.BlockSpec((NC, N_NODES, D), lambda i: (0, 0, 0)),
            pl.BlockSpec((1, N_NODES), lambda i: (0, 0)),
        ],
        out_specs=pl.BlockSpec((N_NODES, D), lambda i: (0, 0)),
        out_shape=jax.ShapeDtypeStruct((N_NODES, D), jnp.float32),
    )(outp, dvis)


def kernel(X, node_idx, hedge_idx, W, b):
    ni3 = node_idx.reshape(NW, NCH, C)
    hi3 = hedge_idx.reshape(NW, NCH, C)
    zeros1d = jnp.zeros((N_NODES,), jnp.float32)
    ones1d = jnp.ones((C,), jnp.float32)
    zeros2d = jnp.zeros((320, D), jnp.float32)

    dvp, dep = _hist()(ni3, hi3, zeros1d, ones1d)
    xs, dvis = _project(X, W, b.reshape(1, D), dvp)
    efp = _edge_agg()(xs, ni3, hi3, zeros2d)
    ef = _edge_scale(efp, dep)
    outp = _node_agg()(ef, ni3, hi3, zeros2d)
    return _finalize(outp, dvis)


# speedup vs baseline: 8.9874x; 1.1691x over previous
import functools

import jax
import jax.numpy as jnp
from jax import lax
from jax.experimental import pallas as pl
from jax.experimental.pallas import tpu as pltpu
from jax.experimental.pallas import tpu_sc as plsc

N_NODES = 10000
N_HEDGES = 5000
NNZ = 320000
D = 128

NC = 2
NS = 16
NW = NC * NS
P = NNZ // NW
C = 80
NCH = P // C

EF_PAD = 5120
OUT_PAD = 10240

@functools.cache
def _mesh():
    return plsc.VectorSubcoreMesh(core_axis_name="c", subcore_axis_name="s",
                                  num_cores=NC, num_subcores=NS)


def _hist_body(ni_hbm, hi_hbm, zeros_hbm, ones_hbm, dv_out, de_out,
               dv_sh, de_sh, idx_v, ones_v):
    c = lax.axis_index("c")
    s = lax.axis_index("s")
    wid = c * NS + s

    @pl.when(s == 0)
    def _():
        pltpu.sync_copy(zeros_hbm, dv_sh)

    @pl.when(s == 1)
    def _():
        pltpu.sync_copy(zeros_hbm.at[pl.ds(0, EF_PAD)], de_sh)

    pltpu.sync_copy(ones_hbm, ones_v)
    pltpu.sync_copy(ni_hbm.at[wid], idx_v)
    plsc.subcore_barrier()

    def nbody(j, carry):
        pltpu.sync_copy(ones_v, dv_sh.at[idx_v.at[j]], add=True)
        return carry

    lax.fori_loop(0, NCH, nbody, 0)

    pltpu.sync_copy(hi_hbm.at[wid], idx_v)

    def hbody(j, carry):
        pltpu.sync_copy(ones_v, de_sh.at[idx_v.at[j]], add=True)
        return carry

    lax.fori_loop(0, NCH, hbody, 0)
    plsc.subcore_barrier()

    @pl.when(s == 0)
    def _():
        pltpu.sync_copy(dv_sh, dv_out.at[c])

    @pl.when(s == 1)
    def _():
        pltpu.sync_copy(de_sh, de_out.at[c])


@functools.cache
def _hist():
    return pl.kernel(
        _hist_body,
        out_type=(jax.ShapeDtypeStruct((NC, N_NODES), jnp.float32),
                  jax.ShapeDtypeStruct((NC, EF_PAD), jnp.float32)),
        mesh=_mesh(),
        scratch_types=[
            pltpu.VMEM_SHARED((N_NODES,), jnp.float32),
            pltpu.VMEM_SHARED((EF_PAD,), jnp.float32),
            pltpu.VMEM((NCH, C), jnp.int32),
            pltpu.VMEM((C,), jnp.float32),
        ],
    )


def _proj_body(x_ref, w_ref, b_ref, dvp_ref, xs_ref, dvis_ref):
    x = x_ref[...]
    w = w_ref[...]
    b = b_ref[...]
    dvp = dvp_ref[...]
    dv = dvp[0] + dvp[1]
    dvis = jnp.where(dv > 0, lax.rsqrt(jnp.maximum(dv, 1e-12)), 0.0)
    xt = jnp.dot(x, w, preferred_element_type=jnp.float32) + b
    xs_ref[...] = xt * dvis[:, None]
    dvis_ref[...] = dvis[None, :]


def _project(X, W, b, dvp):
    return pl.pallas_call(
        _proj_body,
        out_shape=[
            jax.ShapeDtypeStruct((N_NODES, D), jnp.float32),
            jax.ShapeDtypeStruct((1, N_NODES), jnp.float32),
        ],
    )(X, W, b, dvp)


def _gather_scatter_loop(tab_hbm, acc_sh, gi, si, rows0, rows1, sem0, sem1):
    pltpu.async_copy(tab_hbm.at[gi.at[0]], rows0, sem0)

    def body(i, carry):
        j = 2 * i
        pltpu.async_copy(tab_hbm.at[gi.at[j + 1]], rows1, sem1)
        pltpu.make_async_copy(tab_hbm.at[gi.at[j]], rows0, sem0).wait()
        pltpu.sync_copy(rows0, acc_sh.at[si.at[j]], add=True)
        pltpu.async_copy(tab_hbm.at[gi.at[j + 2]], rows0, sem0)
        pltpu.make_async_copy(tab_hbm.at[gi.at[j + 1]], rows1, sem1).wait()
        pltpu.sync_copy(rows1, acc_sh.at[si.at[j + 1]], add=True)
        return carry

    lax.fori_loop(0, (NCH - 1) // 2, body, 0)
    pltpu.make_async_copy(tab_hbm.at[gi.at[NCH - 1]], rows0, sem0).wait()
    pltpu.sync_copy(rows0, acc_sh.at[si.at[NCH - 1]], add=True)


def _edge_body(xs_hbm, ni_hbm, hi_hbm, zeros_hbm, ef_out,
               ef_sh, idxn, idxh, rows0, rows1, sem0, sem1):
    c = lax.axis_index("c")
    s = lax.axis_index("s")
    wid = c * NS + s

    for k in range(5):
        pltpu.sync_copy(zeros_hbm.at[pl.ds(0, 64)],
                        ef_sh.at[pl.ds(s * 320 + k * 64, 64)])
    pltpu.sync_copy(ni_hbm.at[wid], idxn)
    pltpu.sync_copy(hi_hbm.at[wid], idxh)
    plsc.subcore_barrier()
    _gather_scatter_loop(xs_hbm, ef_sh, idxn, idxh, rows0, rows1, sem0, sem1)
    plsc.subcore_barrier()
    for k in range(5):
        pltpu.sync_copy(ef_sh.at[pl.ds(s * 320 + k * 64, 64)],
                        ef_out.at[c, pl.ds(s * 320 + k * 64, 64)])


@functools.cache
def _edge_agg():
    return pl.kernel(
        _edge_body,
        out_type=jax.ShapeDtypeStruct((NC, EF_PAD, D), jnp.float32),
        mesh=_mesh(),
        scratch_types=[
            pltpu.VMEM_SHARED((EF_PAD, D), jnp.float32),
            pltpu.VMEM((NCH, C), jnp.int32),
            pltpu.VMEM((NCH, C), jnp.int32),
            pltpu.VMEM((C, D), jnp.float32),
            pltpu.VMEM((C, D), jnp.float32),
            pltpu.SemaphoreType.DMA,
            pltpu.SemaphoreType.DMA,
        ],
    )


def _escale_body(efp_ref, dep_ref, ef_ref):
    efp = efp_ref[...]
    dep = dep_ref[...]
    de = dep[0] + dep[1]
    dei = jnp.where(de > 0, 1.0 / jnp.maximum(de, 1e-12), 0.0)
    ef_ref[...] = (efp[0] + efp[1]) * dei[:, None]


def _edge_scale(efp, dep):
    return pl.pallas_call(
        _escale_body,
        out_shape=jax.ShapeDtypeStruct((EF_PAD, D), jnp.float32),
    )(efp, dep)


def _node_body(ef_hbm, ni_hbm, hi_hbm, zeros_hbm, out_hbm,
               out_sh, idxn, idxh, rows0, rows1, sem0, sem1):
    c = lax.axis_index("c")
    s = lax.axis_index("s")
    wid = c * NS + s

    for k in range(5):
        pltpu.sync_copy(zeros_hbm.at[pl.ds(0, 128)],
                        out_sh.at[pl.ds(s * 640 + k * 128, 128)])
    pltpu.sync_copy(ni_hbm.at[wid], idxn)
    pltpu.sync_copy(hi_hbm.at[wid], idxh)
    plsc.subcore_barrier()

    def body(j, carry):
        pltpu.async_copy(ef_hbm.at[idxh.at[j]], rows0, sem0).wait()
        pltpu.sync_copy(rows0, out_sh.at[idxn.at[j]], add=True)
        return carry

    lax.fori_loop(0, NCH, body, 0)
    plsc.subcore_barrier()
    for k in range(5):
        pltpu.sync_copy(out_sh.at[pl.ds(s * 640 + k * 128, 128)],
                        out_hbm.at[c, pl.ds(s * 640 + k * 128, 128)])


@functools.cache
def _node_agg():
    return pl.kernel(
        _node_body,
        out_type=jax.ShapeDtypeStruct((NC, OUT_PAD, D), jnp.float32),
        mesh=_mesh(),
        scratch_types=[
            pltpu.VMEM_SHARED((OUT_PAD, D), jnp.float32),
            pltpu.VMEM((NCH, C), jnp.int32),
            pltpu.VMEM((NCH, C), jnp.int32),
            pltpu.VMEM((C, D), jnp.float32),
            pltpu.VMEM((C, D), jnp.float32),
            pltpu.SemaphoreType.DMA,
            pltpu.SemaphoreType.DMA,
        ],
    )


def _final_body(op_ref, dvis_ref, o_ref):
    op = op_ref[...]
    dvis = dvis_ref[...]
    o_ref[...] = jnp.maximum((op[0] + op[1]) * dvis[0][:, None], 0.0)


def _finalize(outp, dvis):
    return pl.pallas_call(
        _final_body,
        grid=(1,),
        in_specs=[
            pl.BlockSpec((NC, N_NODES, D), lambda i: (0, 0, 0)),
            pl.BlockSpec((1, N_NODES), lambda i: (0, 0)),
        ],
        out_specs=pl.BlockSpec((N_NODES, D), lambda i: (0, 0)),
        out_shape=jax.ShapeDtypeStruct((N_NODES, D), jnp.float32),
    )(outp, dvis)


def kernel(X, node_idx, hedge_idx, W, b):
    ni3 = node_idx.reshape(NW, NCH, C)
    hi3 = hedge_idx.reshape(NW, NCH, C)
    zeros1d = jnp.zeros((N_NODES,), jnp.float32)
    ones1d = jnp.ones((C,), jnp.float32)
    zeros2d = jnp.zeros((128, D), jnp.float32)

    dvp, dep = _hist()(ni3, hi3, zeros1d, ones1d)
    xs, dvis = _project(X, W, b.reshape(1, D), dvp)
    efp = _edge_agg()(xs, ni3, hi3, zeros2d)
    ef = _edge_scale(efp, dep)
    outp = _node_agg()(ef, ni3, hi3, zeros2d)
    return _finalize(outp, dvis)


# speedup vs baseline: 9.5846x; 1.0665x over previous
import functools

import jax
import jax.numpy as jnp
from jax import lax
from jax.experimental import pallas as pl
from jax.experimental.pallas import tpu as pltpu
from jax.experimental.pallas import tpu_sc as plsc

N_NODES = 10000
N_HEDGES = 5000
NNZ = 320000
D = 128

NC = 2
NS = 16
NW = NC * NS
P = NNZ // NW
C = 80
NCH = P // C

DE_PAD = 5120


@functools.cache
def _mesh():
    return plsc.VectorSubcoreMesh(core_axis_name="c", subcore_axis_name="s",
                                  num_cores=NC, num_subcores=NS)


def _hist_body(ni_hbm, hi_hbm, zeros_hbm, ones_hbm, dv_out, de_out,
               dv_sh, de_sh, idx_v, ones_v):
    c = lax.axis_index("c")
    s = lax.axis_index("s")
    wid = c * NS + s

    @pl.when(s == 0)
    def _():
        pltpu.sync_copy(zeros_hbm, dv_sh)

    @pl.when(s == 1)
    def _():
        pltpu.sync_copy(zeros_hbm.at[pl.ds(0, DE_PAD)], de_sh)

    pltpu.sync_copy(ones_hbm, ones_v)
    pltpu.sync_copy(ni_hbm.at[wid], idx_v)
    plsc.subcore_barrier()

    def nbody(j, carry):
        pltpu.sync_copy(ones_v, dv_sh.at[idx_v.at[j]], add=True)
        return carry

    lax.fori_loop(0, NCH, nbody, 0)

    pltpu.sync_copy(hi_hbm.at[wid], idx_v)

    def hbody(j, carry):
        pltpu.sync_copy(ones_v, de_sh.at[idx_v.at[j]], add=True)
        return carry

    lax.fori_loop(0, NCH, hbody, 0)
    plsc.subcore_barrier()

    @pl.when(s == 0)
    def _():
        pltpu.sync_copy(dv_sh, dv_out.at[c])

    @pl.when(s == 1)
    def _():
        pltpu.sync_copy(de_sh, de_out.at[c])


@functools.cache
def _hist():
    return pl.kernel(
        _hist_body,
        out_type=(jax.ShapeDtypeStruct((NC, N_NODES), jnp.float32),
                  jax.ShapeDtypeStruct((NC, DE_PAD), jnp.float32)),
        mesh=_mesh(),
        scratch_types=[
            pltpu.VMEM_SHARED((N_NODES,), jnp.float32),
            pltpu.VMEM_SHARED((DE_PAD,), jnp.float32),
            pltpu.VMEM((NCH, C), jnp.int32),
            pltpu.VMEM((C,), jnp.float32),
        ],
    )


def _proj_body(x_ref, w_ref, b_ref, dvp_ref, xs_ref, dvis_ref):
    x = x_ref[...]
    w = w_ref[...]
    b = b_ref[...]
    dvp = dvp_ref[...]
    dv = dvp[0] + dvp[1]
    dvis = jnp.where(dv > 0, lax.rsqrt(jnp.maximum(dv, 1e-12)), 0.0)
    xt = jnp.dot(x, w, preferred_element_type=jnp.float32) + b
    xs_ref[...] = xt * dvis[:, None]
    dvis_ref[...] = dvis[None, :]


def _project(X, W, b, dvp):
    return pl.pallas_call(
        _proj_body,
        out_shape=[
            jax.ShapeDtypeStruct((N_NODES, D), jnp.float32),
            jax.ShapeDtypeStruct((1, N_NODES), jnp.float32),
        ],
    )(X, W, b, dvp)


def _gather_scatter_loop(tab_hbm, acc_sh, gi, si, rows0, rows1, sem0, sem1):
    pltpu.async_copy(tab_hbm.at[gi.at[0]], rows0, sem0)

    def body(i, carry):
        j = 2 * i
        pltpu.async_copy(tab_hbm.at[gi.at[j + 1]], rows1, sem1)
        pltpu.make_async_copy(tab_hbm.at[gi.at[j]], rows0, sem0).wait()
        pltpu.sync_copy(rows0, acc_sh.at[si.at[j]], add=True)
        pltpu.async_copy(tab_hbm.at[gi.at[j + 2]], rows0, sem0)
        pltpu.make_async_copy(tab_hbm.at[gi.at[j + 1]], rows1, sem1).wait()
        pltpu.sync_copy(rows1, acc_sh.at[si.at[j + 1]], add=True)
        return carry

    lax.fori_loop(0, (NCH - 1) // 2, body, 0)
    pltpu.make_async_copy(tab_hbm.at[gi.at[NCH - 1]], rows0, sem0).wait()
    pltpu.sync_copy(rows0, acc_sh.at[si.at[NCH - 1]], add=True)


def _zero_vmem_rows(buf, nrows):
    z = jnp.zeros((16,), jnp.float32)

    def body(r, carry):
        for col in range(D // 16):
            buf[r, pl.ds(col * 16, 16)] = z
        return carry

    lax.fori_loop(0, nrows, body, 0)


def _ranged_copy(mk_src, mk_dst, s, per, total):
    rest = total - (NS - 1) * per

    def chunks(base, n):
        off = 0
        while off < n:
            sz = min(80, n - off)
            pltpu.sync_copy(mk_src(base + off, sz), mk_dst(base + off, sz))
            off += sz

    @pl.when(s < NS - 1)
    def _():
        chunks(s * per, per)

    @pl.when(s == NS - 1)
    def _():
        chunks((NS - 1) * per, rest)


def _edge_body(xs_hbm, ni_hbm, hi_hbm, ef_out,
               ef_sh, idxn, idxh, rows0, rows1, sem0, sem1):
    c = lax.axis_index("c")
    s = lax.axis_index("s")
    wid = c * NS + s

    _zero_vmem_rows(rows0, C)
    _ranged_copy(lambda off, sz: rows0.at[pl.ds(0, sz)],
                 lambda off, sz: ef_sh.at[pl.ds(off, sz)],
                 s, 312, N_HEDGES)
    pltpu.sync_copy(ni_hbm.at[wid], idxn)
    pltpu.sync_copy(hi_hbm.at[wid], idxh)
    plsc.subcore_barrier()
    _gather_scatter_loop(xs_hbm, ef_sh, idxn, idxh, rows0, rows1, sem0, sem1)
    plsc.subcore_barrier()
    _ranged_copy(lambda off, sz: ef_sh.at[pl.ds(off, sz)],
                 lambda off, sz: ef_out.at[c, pl.ds(off, sz)],
                 s, 312, N_HEDGES)


@functools.cache
def _edge_agg():
    return pl.kernel(
        _edge_body,
        out_type=jax.ShapeDtypeStruct((NC, N_HEDGES, D), jnp.float32),
        mesh=_mesh(),
        scratch_types=[
            pltpu.VMEM_SHARED((N_HEDGES, D), jnp.float32),
            pltpu.VMEM((NCH, C), jnp.int32),
            pltpu.VMEM((NCH, C), jnp.int32),
            pltpu.VMEM((C, D), jnp.float32),
            pltpu.VMEM((C, D), jnp.float32),
            pltpu.SemaphoreType.DMA,
            pltpu.SemaphoreType.DMA,
        ],
    )


def _escale_body(efp_ref, dep_ref, ef_ref):
    efp = efp_ref[...]
    dep = dep_ref[...]
    de = (dep[0] + dep[1])[:N_HEDGES]
    dei = jnp.where(de > 0, 1.0 / jnp.maximum(de, 1e-12), 0.0)
    ef_ref[...] = (efp[0] + efp[1]) * dei[:, None]


def _edge_scale(efp, dep):
    return pl.pallas_call(
        _escale_body,
        out_shape=jax.ShapeDtypeStruct((N_HEDGES, D), jnp.float32),
    )(efp, dep)


def _node_body(ef_hbm, ni_hbm, hi_hbm, out_hbm,
               out_sh, idxn, idxh, rows0, rows1, sem0, sem1):
    c = lax.axis_index("c")
    s = lax.axis_index("s")
    wid = c * NS + s

    _zero_vmem_rows(rows0, C)
    _ranged_copy(lambda off, sz: rows0.at[pl.ds(0, sz)],
                 lambda off, sz: out_sh.at[pl.ds(off, sz)],
                 s, 624, N_NODES)
    pltpu.sync_copy(ni_hbm.at[wid], idxn)
    pltpu.sync_copy(hi_hbm.at[wid], idxh)
    plsc.subcore_barrier()

    def body(j, carry):
        pltpu.async_copy(ef_hbm.at[idxh.at[j]], rows0, sem0).wait()
        pltpu.sync_copy(rows0, out_sh.at[idxn.at[j]], add=True)
        return carry

    lax.fori_loop(0, NCH, body, 0)
    plsc.subcore_barrier()
    _ranged_copy(lambda off, sz: out_sh.at[pl.ds(off, sz)],
                 lambda off, sz: out_hbm.at[c, pl.ds(off, sz)],
                 s, 624, N_NODES)


@functools.cache
def _node_agg():
    return pl.kernel(
        _node_body,
        out_type=jax.ShapeDtypeStruct((NC, N_NODES, D), jnp.float32),
        mesh=_mesh(),
        scratch_types=[
            pltpu.VMEM_SHARED((N_NODES, D), jnp.float32),
            pltpu.VMEM((NCH, C), jnp.int32),
            pltpu.VMEM((NCH, C), jnp.int32),
            pltpu.VMEM((C, D), jnp.float32),
            pltpu.VMEM((C, D), jnp.float32),
            pltpu.SemaphoreType.DMA,
            pltpu.SemaphoreType.DMA,
        ],
    )


def _final_body(op_ref, dvis_ref, o_ref):
    op = op_ref[...]
    dvis = dvis_ref[...]
    o_ref[...] = jnp.maximum((op[0] + op[1]) * dvis[0][:, None], 0.0)


def _finalize(outp, dvis):
    return pl.pallas_call(
        _final_body,
        out_shape=jax.ShapeDtypeStruct((N_NODES, D), jnp.float32),
    )(outp, dvis)


def kernel(X, node_idx, hedge_idx, W, b):
    ni3 = node_idx.reshape(NW, NCH, C)
    hi3 = hedge_idx.reshape(NW, NCH, C)

    zeros1d = jnp.zeros((N_NODES,), jnp.float32)
    ones1d = jnp.ones((C,), jnp.float32)
    dvp, dep = _hist()(ni3, hi3, zeros1d, ones1d)
    xs, dvis = _project(X, W, b.reshape(1, D), dvp)
    efp = _edge_agg()(xs, ni3, hi3)
    ef = _edge_scale(efp, dep)
    outp = _node_agg()(ef, ni3, hi3)
    return _finalize(outp, dvis)
